# baseline (speedup 1.0000x reference)
import jax
import jax.numpy as jnp
from jax import lax
from jax.experimental import pallas as pl
from jax.experimental.pallas import tpu as pltpu
from jax.experimental.pallas import tpu_sc as plsc

_SEQ = 200
_DIM = 64
_BATCH = 1024

_NUM_CORES = 2
_NUM_SUBCORES = 16
_NUM_WORKERS = _NUM_CORES * _NUM_SUBCORES
_SEQ_PER_WORKER = _BATCH // _NUM_WORKERS
_TOK_PER_WORKER = _SEQ_PER_WORKER * _SEQ

_CHUNK_ROWS = 800
_SEQ_PER_CHUNK = _CHUNK_ROWS // _SEQ
_NUM_CHUNKS = _TOK_PER_WORKER // _CHUNK_ROWS
_GROUPS_PER_CHUNK = _CHUNK_ROWS // 16
_STREAMS = [(0, 128), (128, 128), (256, 128), (384, 128), (512, 128),
            (640, 128), (768, 32)]
_POS_PAD = 208


def _positional_encoding():
    positions = jnp.arange(_SEQ, dtype=jnp.float32)
    indices = jnp.arange(_DIM // 2, dtype=jnp.float32)
    scaling = 10000.0 ** (2.0 * indices / _DIM)
    angles = positions[:, None] / scaling[None, :]
    pe = jnp.zeros((_SEQ, _DIM), dtype=jnp.float32)
    pe = pe.at[:, 0::2].set(jnp.sin(angles))
    pe = pe.at[:, 1::2].set(jnp.cos(angles))
    return jnp.concatenate([pe, pe[: _POS_PAD - _SEQ]], axis=0)


def _sc_body(x_hbm, table_hbm, pos_hbm, out_hbm, idx_v, rows0, rows1, pos_v,
             gsem0, gsem1, osem0, osem1):
    wid = lax.axis_index("s") * _NUM_CORES + lax.axis_index("c")
    tok_base = wid * _TOK_PER_WORKER
    seq_base = wid * _SEQ_PER_WORKER

    pltpu.sync_copy(pos_hbm, pos_v)
    pltpu.sync_copy(x_hbm.at[pl.ds(tok_base, _TOK_PER_WORKER)], idx_v)

    bufs = (rows0, rows1)
    gsems = (gsem0, gsem1)
    osems = (osem0, osem1)

    def fire_gathers(c, b):
        return [
            pltpu.async_copy(
                table_hbm.at[idx_v.at[pl.ds(c * _CHUNK_ROWS + off, n)]],
                bufs[b].at[pl.ds(off, n)],
                gsems[b],
            )
            for off, n in _STREAMS
        ]

    def fire_outs(c, b):
        return [
            pltpu.async_copy(
                bufs[b].at[pl.ds(s * _SEQ, _SEQ)],
                out_hbm.at[seq_base + c * _SEQ_PER_CHUNK + s],
                osems[b],
            )
            for s in range(_SEQ_PER_CHUNK)
        ]

    def mk_per_group(rows, base):
        def per_group(g, _):
            off = g * 16
            p0 = lax.rem(off, _SEQ)
            idxg = idx_v[pl.ds(base + off, 16)]
            mf = jnp.where(idxg == 0, 0.0, 1.0).astype(jnp.float32)
            for r in range(16):
                m = mf[r]
                t = off + r
                p = p0 + r
                for q in range(_DIM // 16):
                    sl = pl.ds(q * 16, 16)
                    rows[t, sl] = (rows[t, sl] + pos_v[p, sl]) * m
            return 0

        return per_group

    g_desc = fire_gathers(0, 0)
    o_desc = [None, None]
    for c in range(_NUM_CHUNKS):
        b = c % 2
        nb = 1 - b
        if c + 1 < _NUM_CHUNKS:
            if o_desc[nb] is not None:
                for d in o_desc[nb]:
                    d.wait()
            next_g = fire_gathers(c + 1, nb)
        for d in g_desc:
            d.wait()
        lax.fori_loop(0, _GROUPS_PER_CHUNK,
                      mk_per_group(bufs[b], c * _CHUNK_ROWS), 0)
        o_desc[b] = fire_outs(c, b)
        if c + 1 < _NUM_CHUNKS:
            g_desc = next_g
    for bb in range(2):
        for d in o_desc[bb]:
            d.wait()


def kernel(x, table):
    pos = _positional_encoding()
    x = x.astype(jnp.int32).reshape(-1)
    mesh = plsc.VectorSubcoreMesh(core_axis_name="c", subcore_axis_name="s")
    run = pl.kernel(
        _sc_body,
        out_type=jax.ShapeDtypeStruct((_BATCH, _SEQ, _DIM), jnp.float32),
        mesh=mesh,
        scratch_types=[
            pltpu.VMEM((_TOK_PER_WORKER,), jnp.int32),
            pltpu.VMEM((_CHUNK_ROWS, _DIM), jnp.float32),
            pltpu.VMEM((_CHUNK_ROWS, _DIM), jnp.float32),
            pltpu.VMEM((_POS_PAD, _DIM), jnp.float32),
            pltpu.SemaphoreType.DMA,
            pltpu.SemaphoreType.DMA,
            pltpu.SemaphoreType.DMA,
            pltpu.SemaphoreType.DMA,
        ],
        compiler_params=pltpu.CompilerParams(use_tc_tiling_on_sc=False),
    )
    return run(x, table, pos)


# speedup vs baseline: 1.1307x; 1.1307x over previous
import jax
import jax.numpy as jnp
from jax import lax
from jax.experimental import pallas as pl
from jax.experimental.pallas import tpu as pltpu
from jax.experimental.pallas import tpu_sc as plsc

_SEQ = 200
_DIM = 64
_BATCH = 1024

_NUM_CORES = 2
_NUM_SUBCORES = 16
_NUM_WORKERS = _NUM_CORES * _NUM_SUBCORES
_SEQ_PER_WORKER = _BATCH // _NUM_WORKERS
_TOK_PER_WORKER = _SEQ_PER_WORKER * _SEQ

_CHUNK_ROWS = 400
_SEQ_PER_CHUNK = _CHUNK_ROWS // _SEQ
_NUM_CHUNKS = _TOK_PER_WORKER // _CHUNK_ROWS
_GROUPS_PER_CHUNK = _CHUNK_ROWS // 16
_STREAMS = [(0, 128), (128, 128), (256, 128), (384, 16)]
_POS_PAD = 208


def _positional_encoding():
    positions = jnp.arange(_SEQ, dtype=jnp.float32)
    indices = jnp.arange(_DIM // 2, dtype=jnp.float32)
    scaling = 10000.0 ** (2.0 * indices / _DIM)
    angles = positions[:, None] / scaling[None, :]
    pe = jnp.zeros((_SEQ, _DIM), dtype=jnp.float32)
    pe = pe.at[:, 0::2].set(jnp.sin(angles))
    pe = pe.at[:, 1::2].set(jnp.cos(angles))
    return jnp.concatenate([pe, pe[: _POS_PAD - _SEQ]], axis=0)


def _sc_body(x_hbm, table_hbm, pos_hbm, out_hbm, idx_v, in0, in1, out0, out1,
             pos_v, gsem0, gsem1, osem0, osem1):
    wid = lax.axis_index("s") * _NUM_CORES + lax.axis_index("c")
    tok_base = wid * _TOK_PER_WORKER
    seq_base = wid * _SEQ_PER_WORKER

    pltpu.sync_copy(pos_hbm, pos_v)
    pltpu.sync_copy(x_hbm.at[pl.ds(tok_base, _TOK_PER_WORKER)], idx_v)

    def fire_gathers(c, in_buf, gsem):
        for off, n in _STREAMS:
            pltpu.async_copy(
                table_hbm.at[idx_v.at[pl.ds(c * _CHUNK_ROWS + off, n)]],
                in_buf.at[pl.ds(off, n)],
                gsem,
            )

    def drain_gathers(in_buf, gsem):
        for off, n in _STREAMS:
            pltpu.make_async_copy(
                table_hbm.at[idx_v.at[pl.ds(off, n)]],
                in_buf.at[pl.ds(off, n)],
                gsem,
            ).wait()

    def fire_outs(c, out_buf, osem):
        for s in range(_SEQ_PER_CHUNK):
            pltpu.async_copy(
                out_buf.at[pl.ds(s * _SEQ, _SEQ)],
                out_hbm.at[seq_base + c * _SEQ_PER_CHUNK + s],
                osem,
            )

    def drain_outs(out_buf, osem):
        for s in range(_SEQ_PER_CHUNK):
            pltpu.make_async_copy(
                out_buf.at[pl.ds(s * _SEQ, _SEQ)],
                out_hbm.at[seq_base + s],
                osem,
            ).wait()

    def compute(base, in_buf, out_buf):
        def per_group(g, _):
            off = g * 16
            p0 = lax.rem(off, _SEQ)
            idxg = idx_v[pl.ds(base + off, 16)]
            mf = jnp.where(idxg == 0, 0.0, 1.0).astype(jnp.float32)
            for r in range(16):
                m = mf[r]
                t = off + r
                p = p0 + r
                vs = [in_buf[t, pl.ds(q * 16, 16)] for q in range(4)]
                ps = [pos_v[p, pl.ds(q * 16, 16)] for q in range(4)]
                os_ = [(vs[q] + ps[q]) * m for q in range(4)]
                for q in range(4):
                    out_buf[t, pl.ds(q * 16, 16)] = os_[q]
            return 0

        lax.fori_loop(0, _GROUPS_PER_CHUNK, per_group, 0)

    fire_gathers(0, in0, gsem0)

    def body(j, _):
        ca = 2 * j
        cb = 2 * j + 1
        fire_gathers(cb, in1, gsem1)
        drain_gathers(in0, gsem0)

        @pl.when(j > 0)
        def _():
            drain_outs(out0, osem0)

        compute(ca * _CHUNK_ROWS, in0, out0)
        fire_outs(ca, out0, osem0)

        @pl.when(j < _NUM_CHUNKS // 2 - 1)
        def _():
            fire_gathers(ca + 2, in0, gsem0)

        drain_gathers(in1, gsem1)

        @pl.when(j > 0)
        def _():
            drain_outs(out1, osem1)

        compute(cb * _CHUNK_ROWS, in1, out1)
        fire_outs(cb, out1, osem1)
        return 0

    lax.fori_loop(0, _NUM_CHUNKS // 2, body, 0)
    drain_outs(out0, osem0)
    drain_outs(out1, osem1)


def kernel(x, table):
    pos = _positional_encoding()
    x = x.astype(jnp.int32).reshape(-1)
    mesh = plsc.VectorSubcoreMesh(core_axis_name="c", subcore_axis_name="s")
    run = pl.kernel(
        _sc_body,
        out_type=jax.ShapeDtypeStruct((_BATCH, _SEQ, _DIM), jnp.float32),
        mesh=mesh,
        scratch_types=[
            pltpu.VMEM((_TOK_PER_WORKER,), jnp.int32),
            pltpu.VMEM((_CHUNK_ROWS, _DIM), jnp.float32),
            pltpu.VMEM((_CHUNK_ROWS, _DIM), jnp.float32),
            pltpu.VMEM((_CHUNK_ROWS, _DIM), jnp.float32),
            pltpu.VMEM((_CHUNK_ROWS, _DIM), jnp.float32),
            pltpu.VMEM((_POS_PAD, _DIM), jnp.float32),
            pltpu.SemaphoreType.DMA,
            pltpu.SemaphoreType.DMA,
            pltpu.SemaphoreType.DMA,
            pltpu.SemaphoreType.DMA,
        ],
        compiler_params=pltpu.CompilerParams(use_tc_tiling_on_sc=False),
    )
    return run(x, table, pos)


# speedup vs baseline: 1.1778x; 1.0416x over previous
import jax
import jax.numpy as jnp
from jax import lax
from jax.experimental import pallas as pl
from jax.experimental.pallas import tpu as pltpu
from jax.experimental.pallas import tpu_sc as plsc

_SEQ = 200
_DIM = 64
_BATCH = 1024

_NUM_CORES = 2
_NUM_SUBCORES = 16
_NUM_WORKERS = _NUM_CORES * _NUM_SUBCORES
_SEQ_PER_WORKER = _BATCH // _NUM_WORKERS
_TOK_PER_WORKER = _SEQ_PER_WORKER * _SEQ

_HALF_A = 96
_HALF_B = 104


def _positional_encoding():
    positions = jnp.arange(_SEQ, dtype=jnp.float32)
    indices = jnp.arange(_DIM // 2, dtype=jnp.float32)
    scaling = 10000.0 ** (2.0 * indices / _DIM)
    angles = positions[:, None] / scaling[None, :]
    pe = jnp.zeros((_SEQ, _DIM), dtype=jnp.float32)
    pe = pe.at[:, 0::2].set(jnp.sin(angles))
    pe = pe.at[:, 1::2].set(jnp.cos(angles))
    return pe


def _sc_body(x_hbm, table_hbm, pos_hbm, out_hbm, idx_v, pidx0, pidx1, in0,
             in1, out0, out1, pos_v, gsem0, gsem1, osem0, osem1):
    wid = lax.axis_index("s") * _NUM_CORES + lax.axis_index("c")
    tok_base = wid * _TOK_PER_WORKER
    seq_base = wid * _SEQ_PER_WORKER

    pltpu.sync_copy(pos_hbm, pos_v)
    pltpu.sync_copy(x_hbm.at[pl.ds(tok_base, _TOK_PER_WORKER)],
                    idx_v.at[pl.ds(0, _TOK_PER_WORKER)])

    def fire_gathers(abs_base, n_rows, pidx, in_buf, gsem):
        def fill(i, _):
            v = idx_v[pl.ds(abs_base + i * 16, 16)]
            pidx[pl.ds(i * 16, 16)] = lax.shift_right_logical(v, 1)
            return 0

        lax.fori_loop(0, (n_rows + 15) // 16, fill, 0)
        pltpu.async_copy(
            table_hbm.at[pidx.at[pl.ds(0, n_rows)]],
            in_buf.at[pl.ds(0, n_rows)],
            gsem,
        )

    def drain_gathers(n_rows, pidx, in_buf, gsem):
        pltpu.make_async_copy(
            table_hbm.at[pidx.at[pl.ds(0, n_rows)]],
            in_buf.at[pl.ds(0, n_rows)],
            gsem,
        ).wait()

    def fire_out(seq, pos0, n_rows, out_buf, osem):
        pltpu.async_copy(out_buf, out_hbm.at[seq_base + seq, pl.ds(pos0, n_rows)],
                         osem)

    def drain_out(pos0, n_rows, out_buf, osem):
        pltpu.make_async_copy(out_buf,
                              out_hbm.at[seq_base, pl.ds(pos0, n_rows)],
                              osem).wait()

    def compute(abs_base, pos0, full_groups, peel, in_buf, out_buf):
        def token_block(off, r_range):
            idxg = idx_v[pl.ds(abs_base + off, 16)]
            mf = jnp.where(idxg == 0, 0.0, 1.0).astype(jnp.float32)
            hf = jnp.bitwise_and(idxg, 1) * _DIM
            for r in r_range:
                m = mf[r]
                h = hf[r]
                t = off + r
                vs = [in_buf[t, pl.ds(h + q * 16, 16)] for q in range(4)]
                ps = [pos_v[pos0 + t, pl.ds(q * 16, 16)] for q in range(4)]
                res = [(vs[q] + ps[q]) * m for q in range(4)]
                for q in range(4):
                    out_buf[t, pl.ds(q * 16, 16)] = res[q]

        def per_group(g, _):
            token_block(g * 16, range(16))
            return 0

        lax.fori_loop(0, full_groups, per_group, 0)
        if peel:
            token_block(full_groups * 16, range(peel))

    fire_gathers(0, _HALF_A, pidx0, in0, gsem0)

    def body(j, _):
        base = j * _SEQ
        fire_gathers(base + _HALF_A, _HALF_B, pidx1, in1, gsem1)
        drain_gathers(_HALF_A, pidx0, in0, gsem0)

        @pl.when(j > 0)
        def _():
            drain_out(0, _HALF_A, out0, osem0)

        compute(base, 0, _HALF_A // 16, 0, in0, out0)
        fire_out(j, 0, _HALF_A, out0, osem0)

        @pl.when(j < _SEQ_PER_WORKER - 1)
        def _():
            fire_gathers(base + _SEQ, _HALF_A, pidx0, in0, gsem0)

        drain_gathers(_HALF_B, pidx1, in1, gsem1)

        @pl.when(j > 0)
        def _():
            drain_out(_HALF_A, _HALF_B, out1, osem1)

        compute(base + _HALF_A, _HALF_A, _HALF_B // 16, 8, in1, out1)
        fire_out(j, _HALF_A, _HALF_B, out1, osem1)
        return 0

    lax.fori_loop(0, _SEQ_PER_WORKER, body, 0)
    drain_out(0, _HALF_A, out0, osem0)
    drain_out(_HALF_A, _HALF_B, out1, osem1)


def kernel(x, table):
    pos = _positional_encoding()
    x = x.astype(jnp.int32).reshape(-1)
    tp = table.reshape(500000, 2 * _DIM)
    mesh = plsc.VectorSubcoreMesh(core_axis_name="c", subcore_axis_name="s")
    run = pl.kernel(
        _sc_body,
        out_type=jax.ShapeDtypeStruct((_BATCH, _SEQ, _DIM), jnp.float32),
        mesh=mesh,
        scratch_types=[
            pltpu.VMEM((_TOK_PER_WORKER + 16,), jnp.int32),
            pltpu.VMEM((_HALF_A + 16,), jnp.int32),
            pltpu.VMEM((_HALF_B + 16,), jnp.int32),
            pltpu.VMEM((_HALF_A, 2 * _DIM), jnp.float32),
            pltpu.VMEM((_HALF_B, 2 * _DIM), jnp.float32),
            pltpu.VMEM((_HALF_A, _DIM), jnp.float32),
            pltpu.VMEM((_HALF_B, _DIM), jnp.float32),
            pltpu.VMEM((_SEQ, _DIM), jnp.float32),
            pltpu.SemaphoreType.DMA,
            pltpu.SemaphoreType.DMA,
            pltpu.SemaphoreType.DMA,
            pltpu.SemaphoreType.DMA,
        ],
    )
    return run(x, tp, pos)
